# baseline (speedup 1.0000x reference)
import functools

import jax
import jax.numpy as jnp
from jax import lax
from jax.experimental import pallas as pl
from jax.experimental.pallas import tpu as pltpu
from jax.experimental.pallas import tpu_sc as plsc

_TOPK = 5
_ALPHA = 0.1
_EPS2 = 1e-24


def _sum_body(x_ref, o_ref):
    @pl.when(pl.program_id(0) == 0)
    def _init():
        o_ref[...] = jnp.zeros_like(o_ref)

    o_ref[...] += jnp.sum(x_ref[...], axis=1)


def _scores_body(q_ref, p_ref, s_ref, *, seq_len):
    q = q_ref[...] / seq_len
    qn = q * lax.rsqrt(jnp.maximum(jnp.sum(q * q, axis=1, keepdims=True), _EPS2))
    p = p_ref[...]
    pn2 = jnp.sum(p * p, axis=1)
    dots = lax.dot_general(
        qn, p, (((1,), (1,)), ((), ())), preferred_element_type=jnp.float32
    )
    s_ref[...] = dots * lax.rsqrt(jnp.maximum(pn2, _EPS2))[None, :]


def _topk_weights(s):
    bsz, k = s.shape
    colid = lax.broadcasted_iota(jnp.int32, (bsz, k), 1)
    work = s
    vals, idxs = [], []
    for _ in range(_TOPK):
        m = jnp.max(work, axis=1, keepdims=True)
        idx = jnp.min(jnp.where(work == m, colid, k), axis=1, keepdims=True)
        vals.append(m)
        idxs.append(idx)
        work = jnp.where(colid == idx, -jnp.inf, work)
    v = jnp.concatenate(vals, axis=1)
    e = jnp.exp(v - v[:, :1])
    w = e / jnp.sum(e, axis=1, keepdims=True)
    return w, idxs


def _agg_body(s_ref, p_ref, o_ref, *, kb):
    i = pl.program_id(0)

    @pl.when(i == 0)
    def _init():
        o_ref[...] = jnp.zeros_like(o_ref)

    w, idxs = _topk_weights(s_ref[...])
    bsz = w.shape[0]
    colid = i * kb + lax.broadcasted_iota(jnp.int32, (bsz, kb), 1)
    wblk = jnp.zeros((bsz, kb), jnp.float32)
    for j in range(_TOPK):
        wblk = wblk + jnp.where(colid == idxs[j], w[:, j : j + 1], 0.0)
    o_ref[...] += jnp.dot(wblk, p_ref[...], preferred_element_type=jnp.float32)


def _sc_retrieval(scores, prototypes):
    bsz, k = scores.shape
    nproto, d = prototypes.shape
    nc, ns, nl = 2, 16, 16
    cpb = (nc * ns) // bsz
    chunk = k // cpb
    nv = chunk // nl
    neg = jnp.float32(-3.0e38)

    mesh = plsc.VectorSubcoreMesh(core_axis_name="c", subcore_axis_name="s")
    _dnums = lax.GatherDimensionNumbers(
        offset_dims=(), collapsed_slice_dims=(0,), start_index_map=(0,))

    def _pg(arr, perm):
        return lax.gather(arr, perm[:, None], _dnums, (1,),
                          mode=lax.GatherScatterMode.PROMISE_IN_BOUNDS)

    lanes_c = lax.broadcasted_iota(jnp.int32, (nl,), 0)

    def _bfly_argmax(mv, mi, lanes):
        for st in (8, 4, 2, 1):
            perm = lanes ^ st
            gv = _pg(mv, perm)
            gi = _pg(mi, perm)
            t = (gv > mv) | ((gv == mv) & (gi < mi))
            mv = jnp.where(t, gv, mv)
            mi = jnp.where(t, gi, mi)
        return mv, mi

    def _top5(read_vec, n_iters, unroll, lanes):
        vals, idxs = [], []
        for _ in range(_TOPK):
            excl = list(idxs)

            def step(i, carry, excl=excl):
                mv, mi = carry
                v, vi = read_vec(i)
                for e in excl:
                    v = jnp.where(vi == e, neg, v)
                t = (v > mv) | ((v == mv) & (vi < mi))
                return jnp.where(t, v, mv), jnp.where(t, vi, mi)

            init = (jnp.full((nl,), neg, jnp.float32),
                    jnp.full((nl,), jnp.int32(2**30)))
            mv, mi = lax.fori_loop(0, n_iters, step, init, unroll=unroll)
            mv, mi = _bfly_argmax(mv, mi, lanes)
            vals.append(mv)
            idxs.append(mi)
        return vals, idxs

    @functools.partial(
        pl.kernel,
        out_type=jax.ShapeDtypeStruct((nc * ns, 2, nl), jnp.float32),
        mesh=mesh,
        scratch_types=dict(
            s_v=pltpu.VMEM((chunk,), jnp.float32),
            top2_v=pltpu.VMEM((1, 2, nl), jnp.float32),
        ),
    )
    def scan_body(scores_hbm, cand_hbm, s_v, top2_v):
        c = lax.axis_index("c")
        s = lax.axis_index("s")
        b = c * (ns // cpb) + s // cpb
        ch = s % cpb
        pltpu.sync_copy(scores_hbm.at[pl.ds(b * k + ch * chunk, chunk)], s_v)
        lanes = lax.broadcasted_iota(jnp.int32, (nl,), 0)
        base = ch * chunk

        def read_chunk(i):
            return s_v[pl.ds(i * nl, nl)], base + i * nl + lanes

        vals, idxs = _top5(read_chunk, nv, False, lanes)
        tv = jnp.full((nl,), neg, jnp.float32)
        ti = jnp.zeros((nl,), jnp.float32)
        for r in range(_TOPK):
            tv = jnp.where(lanes == r, vals[r], tv)
            ti = jnp.where(lanes == r, idxs[r].astype(jnp.float32), ti)
        top2_v[0, 0, :] = tv
        top2_v[0, 1, :] = ti
        pltpu.sync_copy(top2_v, cand_hbm.at[pl.ds(b * cpb + ch, 1)])

    @functools.partial(
        pl.kernel,
        out_type=jax.ShapeDtypeStruct((bsz, d), jnp.float32),
        mesh=mesh,
        scratch_types=dict(
            cand_v=pltpu.VMEM((cpb, 2, nl), jnp.float32),
            rows_v=pltpu.VMEM((_TOPK, d), jnp.float32),
            acc_v=pltpu.VMEM((d,), jnp.float32),
        ),
    )
    def merge_body(cand_hbm, protos_hbm, out_hbm, cand_v, rows_v, acc_v):
        c = lax.axis_index("c")
        s = lax.axis_index("s")

        @pl.when((c == 0) & (s < bsz))
        def _merge_and_aggregate():
            b = s
            lanes = lax.broadcasted_iota(jnp.int32, (nl,), 0)
            pltpu.sync_copy(cand_hbm.at[pl.ds(b * cpb, cpb)], cand_v)

            def read_cand(i):
                return cand_v[i, 0, :], cand_v[i, 1, :].astype(jnp.int32)

            vals, idxs = _top5(read_cand, cpb, True, lanes)
            es = [jnp.exp(vals[r] - vals[0]) for r in range(_TOPK)]
            tot = es[0]
            for r in range(1, _TOPK):
                tot = tot + es[r]
            ws = [es[r] / tot for r in range(_TOPK)]
            for r in range(_TOPK):
                pltpu.sync_copy(protos_hbm.at[pl.ds(idxs[r][0], 1)],
                                rows_v.at[pl.ds(r, 1)])

            def acc_step(cc, _):
                sl = pl.ds(cc * nl, nl)
                a = ws[0] * rows_v[0, sl]
                for r in range(1, _TOPK):
                    a = a + ws[r] * rows_v[r, sl]
                acc_v[sl] = a
                return 0

            lax.fori_loop(0, d // nl, acc_step, 0)
            pltpu.sync_copy(acc_v, out_hbm.at[b])

    cand = scan_body(scores.reshape(-1))
    return merge_body(cand, prototypes)


def _add_body(x_ref, a_ref, o_ref):
    o_ref[...] = x_ref[...] + _ALPHA * a_ref[...][:, None, :]


def _pipeline(x, prototypes, interpret=False):
    bsz, seq_len, d = x.shape
    k = prototypes.shape[0]
    lb = 512
    kb = 1024

    qsum = pl.pallas_call(
        _sum_body,
        grid=(seq_len // lb,),
        in_specs=[pl.BlockSpec((bsz, lb, d), lambda i: (0, i, 0))],
        out_specs=pl.BlockSpec((bsz, d), lambda i: (0, 0)),
        out_shape=jax.ShapeDtypeStruct((bsz, d), jnp.float32),
        interpret=interpret,
    )(x)

    scores = pl.pallas_call(
        functools.partial(_scores_body, seq_len=seq_len),
        grid=(k // kb,),
        in_specs=[
            pl.BlockSpec((bsz, d), lambda i: (0, 0)),
            pl.BlockSpec((kb, d), lambda i: (i, 0)),
        ],
        out_specs=pl.BlockSpec((bsz, kb), lambda i: (0, i)),
        out_shape=jax.ShapeDtypeStruct((bsz, k), jnp.float32),
        interpret=interpret,
    )(qsum, prototypes)

    if interpret:
        agg = pl.pallas_call(
            functools.partial(_agg_body, kb=kb),
            grid=(k // kb,),
            in_specs=[
                pl.BlockSpec((bsz, k), lambda i: (0, 0)),
                pl.BlockSpec((kb, d), lambda i: (i, 0)),
            ],
            out_specs=pl.BlockSpec((bsz, d), lambda i: (0, 0)),
            out_shape=jax.ShapeDtypeStruct((bsz, d), jnp.float32),
            interpret=interpret,
        )(scores, prototypes)
    else:
        agg = _sc_retrieval(scores, prototypes)

    out = pl.pallas_call(
        _add_body,
        grid=(seq_len // lb,),
        in_specs=[
            pl.BlockSpec((bsz, lb, d), lambda i: (0, i, 0)),
            pl.BlockSpec((bsz, d), lambda i: (0, 0)),
        ],
        out_specs=pl.BlockSpec((bsz, lb, d), lambda i: (0, i, 0)),
        out_shape=jax.ShapeDtypeStruct((bsz, seq_len, d), jnp.float32),
        interpret=interpret,
    )(x, agg)
    return out


def kernel(x, prototypes):
    return _pipeline(x, prototypes)
